# baseline (speedup 1.0000x reference)
import functools

import jax
import jax.numpy as jnp
from jax import lax
from jax.experimental import pallas as pl
from jax.experimental.pallas import tpu as pltpu
from jax.experimental.pallas import tpu_sc as plsc

ROWS = 64
N = 8192
K = 100
L = 16
NVREG = N // L
I32MIN = -2147483648


def _c(v):
    return jnp.full((L,), v, jnp.int32)


def _splat(s):
    return jnp.full((L,), s, jnp.int32)


def _sort16d(v):
    return lax.rev(lax.sort(v, dimension=0), (0,))


def _bmerge_desc(vs):
    if len(vs) == 1:
        return [_sort16d(vs[0])]
    half = len(vs) // 2
    his, los = [], []
    for i in range(half):
        his.append(jnp.maximum(vs[i], vs[i + half]))
        los.append(jnp.minimum(vs[i], vs[i + half]))
    return _bmerge_desc(his) + _bmerge_desc(los)


def _merge_desc(a, b):
    m = len(a)
    revb = [lax.rev(v, (0,)) for v in b[::-1]]
    his, los = [], []
    for i in range(m):
        his.append(jnp.maximum(a[i], revb[i]))
        los.append(jnp.minimum(a[i], revb[i]))
    return _bmerge_desc(his) + _bmerge_desc(los)


def _sort128_desc(vs):
    runs = [[_sort16d(v)] for v in vs]
    while len(runs) > 1:
        runs = [_merge_desc(runs[i], runs[i + 1]) for i in range(0, len(runs), 2)]
    return runs[0]


def _keyxform(b):
    s = lax.shift_right_arithmetic(b, _c(31))
    return b ^ (s & _c(0x7FFFFFFF))


def _append(ref, off_splat, x, m):
    mi = m.astype(jnp.int32)
    cs = plsc.cumsum(mi)
    idx = off_splat + cs - _c(1)
    plsc.store_scatter(ref, [idx], x, mask=m)
    return off_splat + plsc.all_reduce_population_count(m)


def _select_top(hist, k):
    tail = jnp.int32(0)
    cnt = _c(0)
    ksplat = _splat(k)
    for j in range(15, -1, -1):
        h = hist[pl.ds(16 * j, 16)]
        rc = lax.rev(plsc.cumsum(lax.rev(h, (0,))), (0,))
        suf = rc + _splat(tail)
        cnt = cnt + plsc.all_reduce_population_count(suf >= ksplat)
        tail = tail + jnp.sum(h)
    b = jnp.max(cnt) - jnp.int32(1)
    g = jnp.int32(0)
    bs = _splat(b)
    iota = lax.iota(jnp.int32, L)
    for j in range(16):
        h = hist[pl.ds(16 * j, 16)]
        ids = iota + _c(16 * j)
        g = g + jnp.sum(jnp.where(ids > bs, h, _c(0)))
    return b, g


def _select_bot(hist, k):
    head = jnp.int32(0)
    cnt = _c(0)
    ksplat = _splat(k)
    for j in range(16):
        h = hist[pl.ds(16 * j, 16)]
        pc = plsc.cumsum(h)
        pre = pc + _splat(head)
        cnt = cnt + plsc.all_reduce_population_count(pre >= ksplat)
        head = head + jnp.sum(h)
    b = jnp.int32(256) - jnp.max(cnt)
    lcount = jnp.int32(0)
    bs = _splat(b)
    iota = lax.iota(jnp.int32, L)
    for j in range(16):
        h = hist[pl.ds(16 * j, 16)]
        ids = iota + _c(16 * j)
        lcount = lcount + jnp.sum(jnp.where(ids < bs, h, _c(0)))
    return b, lcount


def _clear_hist(hist):
    z = _c(0)
    for j in range(16):
        hist[pl.ds(16 * j, 16)] = z


def _chowder_body(scores_hbm, out_hbm, rawb, cand_t0, cand_t1, cand_b0,
                  cand_b1, hist, sgb, slb, stage, dsem, osem):
    nc = 2
    wid = lax.axis_index("s") * nc + lax.axis_index("c")
    iota = lax.iota(jnp.int32, L)
    tbufs = [cand_t0, cand_t1]
    bbufs = [cand_b0, cand_b1]

    pltpu.async_copy(scores_hbm.at[wid * 2], rawb.at[0], dsem)

    def row_work(r, carry):
        row = wid * 2 + r
        with jax.named_scope("ph_dma_in"):
            pltpu.make_async_copy(scores_hbm.at[row], rawb.at[r], dsem).wait()

            @pl.when(r == 0)
            def _():
                pltpu.async_copy(scores_hbm.at[wid * 2 + 1], rawb.at[1], dsem)

        _clear_hist(hist)
        ones = _c(1)

        def p1_body(i):
            base = i * 16
            x = rawb[r, pl.ds(base, 16)]
            key = _keyxform(lax.bitcast_convert_type(x, jnp.int32))
            byte0 = lax.shift_right_logical(key, _c(24)) ^ _c(0x80)
            plsc.addupdate_scatter(hist, [byte0], ones)

        with jax.named_scope("ph_pass1"):
            plsc.parallel_loop(0, NVREG, 1, unroll=8)(p1_body)

        with jax.named_scope("ph_select1"):
            b1t, g1 = _select_top(hist, jnp.int32(K))
            b1b, l1 = _select_bot(hist, jnp.int32(K))
        kt = jnp.int32(K) - g1
        kb = jnp.int32(K) - l1

        lt = lax.shift_left(b1t ^ jnp.int32(128), jnp.int32(24))
        ub = lax.shift_left((b1b ^ jnp.int32(128)) + jnp.int32(1),
                            jnp.int32(24)) - jnp.int32(1)
        lts, ubs = _splat(lt), _splat(ub)

        def s2_body(i, offs):
            offt, offb = offs
            base = i * 16
            x = rawb[r, pl.ds(base, 16)]
            key = _keyxform(lax.bitcast_convert_type(x, jnp.int32))
            offt = _append(cand_t0, offt, key, key >= lts)
            offb = _append(cand_b0, offb, key, key <= ubs)
            return offt, offb

        with jax.named_scope("ph_scan2"):
            offt, offb = plsc.parallel_loop(
                0, NVREG, 1, unroll=8, carry=(_c(0), _c(0)))(s2_body)
        nt = offt[0]
        nb = offb[0]

        def refine(bufs, n0, p1byte, k0, is_top):
            p = p1byte ^ jnp.int32(128)
            k = k0
            n = n0
            for lvl in (2, 3, 4):
                src = bufs[lvl % 2]
                dst = bufs[1 - lvl % 2]
                shift_pref = 32 - 8 * (lvl - 1)
                shift_byte = 32 - 8 * lvl
                _clear_hist(hist)
                ps = _splat(p)
                ns = _splat(n)
                trip = (n + jnp.int32(15)) // jnp.int32(16)

                def hbody(i, _sp=shift_pref, _sb=shift_byte, _ps=ps, _ns=ns,
                          _src=src):
                    base = i * 16
                    ch = _src[pl.ds(base, 16)]
                    valid = (iota + _splat(base)) < _ns
                    pref = lax.shift_right_logical(ch, _c(_sp))
                    m_eq = (pref == _ps) & valid
                    byt = lax.shift_right_logical(ch, _c(_sb)) & _c(0xFF)
                    plsc.addupdate_scatter(hist, [byt], ones, mask=m_eq)

                plsc.parallel_loop(0, trip, 1, unroll=4)(hbody)
                if is_top:
                    b, g = _select_top(hist, k)
                else:
                    b, g = _select_bot(hist, k)
                k = k - g
                p = lax.shift_left(p, jnp.int32(8)) | b
                if is_top:
                    bound = lax.shift_left(p, jnp.int32(shift_byte))
                else:
                    bound = lax.shift_left(p + jnp.int32(1),
                                           jnp.int32(shift_byte)) - jnp.int32(1)
                bnd = _splat(bound)

                def fbody(i, off, _bnd=bnd, _ns=ns, _src=src, _dst=dst,
                          _top=is_top):
                    base = i * 16
                    ch = _src[pl.ds(base, 16)]
                    valid = (iota + _splat(base)) < _ns
                    m = ((ch >= _bnd) if _top else (ch <= _bnd)) & valid
                    return _append(_dst, off, ch, m)

                offv = plsc.parallel_loop(
                    0, trip, 1, unroll=4, carry=_c(0))(fbody)
                n = offv[0]
            return p, k, n

        with jax.named_scope("ph_refine"):
            t_top, ties_t, ntf = refine(tbufs, nt, b1t, kt, True)
            t_bot, ties_b, nbf = refine(bbufs, nb, b1b, kb, False)

        neg = jnp.full((L,), I32MIN, jnp.int32)
        for j in range(8):
            sgb[pl.ds(16 * j, 16)] = neg
            slb[pl.ds(16 * j, 16)] = neg

        tts = _splat(t_top)
        tbs = _splat(t_bot)
        ft = tbufs[1]
        fb = bbufs[1]

        def gbody(i, off, _ns=_splat(ntf)):
            base = i * 16
            ch = ft[pl.ds(base, 16)]
            valid = (iota + _splat(base)) < _ns
            return _append(sgb, off, ch, (ch > tts) & valid)

        def lbody(i, off, _ns=_splat(nbf)):
            base = i * 16
            ch = fb[pl.ds(base, 16)]
            valid = (iota + _splat(base)) < _ns
            return _append(slb, off, ch, (ch < tbs) & valid)

        with jax.named_scope("ph_strict"):
            tript = (ntf + jnp.int32(15)) // jnp.int32(16)
            tripb = (nbf + jnp.int32(15)) // jnp.int32(16)
            plsc.parallel_loop(0, tript, 1, unroll=4, carry=_c(0))(gbody)
            plsc.parallel_loop(0, tripb, 1, unroll=4, carry=_c(0))(lbody)

        with jax.named_scope("ph_sort"):
            sg = _sort128_desc([sgb[pl.ds(16 * j, 16)] for j in range(8)])
            sl = _sort128_desc([slb[pl.ds(16 * j, 16)] for j in range(8)])
            for j in range(8):
                slb[pl.ds(16 * j, 16)] = sl[j]

        with jax.named_scope("ph_assemble"):
            ng = _splat(jnp.int32(K) - ties_t)
            tbv = _splat(ties_b)
            for base in (0, 16, 32, 48, 64, 80, 96, 112, 128, 144, 160, 176,
                         184):
                pvec = iota + _c(base)
                if base <= 96:
                    vt = sg[base // 16]
                    valt = jnp.where(pvec < ng, vt, tts)
                if base >= 96:
                    q = pvec - _c(K)
                    qm = jnp.maximum(q - tbv, _c(0))
                    vb = plsc.load_gather(slb, [qm])
                    valb = jnp.where(q < tbv, tbs, vb)
                if base < 96:
                    val = valt
                elif base == 96:
                    val = jnp.where(pvec < _c(K), valt, valb)
                else:
                    val = valb
                stage[r, pl.ds(base, 16)] = lax.bitcast_convert_type(
                    _keyxform(val), jnp.float32)

            pltpu.async_copy(stage.at[r], out_hbm.at[row], osem)
        return carry

    lax.fori_loop(0, 2, row_work, jnp.int32(0))
    for i in range(2):
        pltpu.make_async_copy(stage.at[i], out_hbm.at[wid * 2 + i], osem).wait()


_mesh = plsc.VectorSubcoreMesh(core_axis_name="c", subcore_axis_name="s")

_chowder = functools.partial(
    pl.kernel,
    out_type=jax.ShapeDtypeStruct((ROWS, 2 * K), jnp.float32),
    mesh=_mesh,
    compiler_params=pltpu.CompilerParams(needs_layout_passes=False),
    scratch_types=[
        pltpu.VMEM((2, N), jnp.float32),
        pltpu.VMEM((N + 16,), jnp.int32),
        pltpu.VMEM((N + 16,), jnp.int32),
        pltpu.VMEM((N + 16,), jnp.int32),
        pltpu.VMEM((N + 16,), jnp.int32),
        pltpu.VMEM((256,), jnp.int32),
        pltpu.VMEM((128,), jnp.int32),
        pltpu.VMEM((128,), jnp.int32),
        pltpu.VMEM((2, 2 * K), jnp.float32),
        pltpu.SemaphoreType.DMA,
        pltpu.SemaphoreType.DMA,
    ],
)(_chowder_body)


def kernel(features, chowder_scores):
    del features
    return _chowder(chowder_scores)


# speedup vs baseline: 1.0090x; 1.0090x over previous
import functools

import jax
import jax.numpy as jnp
from jax import lax
from jax.experimental import pallas as pl
from jax.experimental.pallas import tpu as pltpu
from jax.experimental.pallas import tpu_sc as plsc

ROWS = 64
N = 8192
K = 100
L = 16
NVREG = N // L
I32MIN = -2147483648


def _c(v):
    return jnp.full((L,), v, jnp.int32)


def _splat(s):
    return jnp.full((L,), s, jnp.int32)


def _sort16d(v):
    return lax.rev(lax.sort(v, dimension=0), (0,))


def _bmerge_desc(vs):
    if len(vs) == 1:
        return [_sort16d(vs[0])]
    half = len(vs) // 2
    his, los = [], []
    for i in range(half):
        his.append(jnp.maximum(vs[i], vs[i + half]))
        los.append(jnp.minimum(vs[i], vs[i + half]))
    return _bmerge_desc(his) + _bmerge_desc(los)


def _merge_desc(a, b):
    m = len(a)
    revb = [lax.rev(v, (0,)) for v in b[::-1]]
    his, los = [], []
    for i in range(m):
        his.append(jnp.maximum(a[i], revb[i]))
        los.append(jnp.minimum(a[i], revb[i]))
    return _bmerge_desc(his) + _bmerge_desc(los)


def _sort128_desc(vs):
    runs = [[_sort16d(v)] for v in vs]
    while len(runs) > 1:
        runs = [_merge_desc(runs[i], runs[i + 1]) for i in range(0, len(runs), 2)]
    return runs[0]


def _keyxform(b):
    s = lax.shift_right_arithmetic(b, _c(31))
    return b ^ (s & _c(0x7FFFFFFF))


def _append(ref, off_splat, x, m):
    mi = m.astype(jnp.int32)
    cs = plsc.cumsum(mi)
    idx = off_splat + cs - _c(1)
    plsc.store_scatter(ref, [idx], x, mask=m)
    return off_splat + plsc.all_reduce_population_count(m)


def _select_top(hist, k):
    tail = jnp.int32(0)
    cnt = _c(0)
    ksplat = _splat(k)
    for j in range(15, -1, -1):
        h = hist[pl.ds(16 * j, 16)]
        rc = lax.rev(plsc.cumsum(lax.rev(h, (0,))), (0,))
        suf = rc + _splat(tail)
        cnt = cnt + plsc.all_reduce_population_count(suf >= ksplat)
        tail = tail + jnp.sum(h)
    b = jnp.max(cnt) - jnp.int32(1)
    g = jnp.int32(0)
    bs = _splat(b)
    iota = lax.iota(jnp.int32, L)
    for j in range(16):
        h = hist[pl.ds(16 * j, 16)]
        ids = iota + _c(16 * j)
        g = g + jnp.sum(jnp.where(ids > bs, h, _c(0)))
    return b, g


def _select_bot(hist, k):
    head = jnp.int32(0)
    cnt = _c(0)
    ksplat = _splat(k)
    for j in range(16):
        h = hist[pl.ds(16 * j, 16)]
        pc = plsc.cumsum(h)
        pre = pc + _splat(head)
        cnt = cnt + plsc.all_reduce_population_count(pre >= ksplat)
        head = head + jnp.sum(h)
    b = jnp.int32(256) - jnp.max(cnt)
    lcount = jnp.int32(0)
    bs = _splat(b)
    iota = lax.iota(jnp.int32, L)
    for j in range(16):
        h = hist[pl.ds(16 * j, 16)]
        ids = iota + _c(16 * j)
        lcount = lcount + jnp.sum(jnp.where(ids < bs, h, _c(0)))
    return b, lcount


def _clear_hist(hist):
    z = _c(0)
    for j in range(16):
        hist[pl.ds(16 * j, 16)] = z


def _chowder_body(scores_hbm, out_hbm, rawb, cand_t0, cand_t1, cand_b0,
                  cand_b1, hist, sgb, slb, stage, dsem, osem):
    nc = 2
    wid = lax.axis_index("s") * nc + lax.axis_index("c")
    iota = lax.iota(jnp.int32, L)
    tbufs = [cand_t0, cand_t1]
    bbufs = [cand_b0, cand_b1]

    pltpu.async_copy(scores_hbm.at[wid * 2], rawb.at[0], dsem)

    def row_work(r, carry):
        row = wid * 2 + r
        pltpu.make_async_copy(scores_hbm.at[row], rawb.at[r], dsem).wait()

        @pl.when(r == 0)
        def _():
            pltpu.async_copy(scores_hbm.at[wid * 2 + 1], rawb.at[1], dsem)

        _clear_hist(hist)
        ones = _c(1)

        def p1_body(i):
            base = i * 16
            x = rawb[r, pl.ds(base, 16)]
            key = _keyxform(lax.bitcast_convert_type(x, jnp.int32))
            byte0 = lax.shift_right_logical(key, _c(24)) ^ _c(0x80)
            plsc.addupdate_scatter(hist, [byte0], ones)

        plsc.parallel_loop(0, NVREG, 1, unroll=8)(p1_body)

        b1t, g1 = _select_top(hist, jnp.int32(K))
        b1b, l1 = _select_bot(hist, jnp.int32(K))
        kt = jnp.int32(K) - g1
        kb = jnp.int32(K) - l1

        lt = lax.shift_left(b1t ^ jnp.int32(128), jnp.int32(24))
        ub = lax.shift_left((b1b ^ jnp.int32(128)) + jnp.int32(1),
                            jnp.int32(24)) - jnp.int32(1)
        lts, ubs = _splat(lt), _splat(ub)

        def s2_body(i, offs):
            offt, offb = offs
            base = i * 16
            x = rawb[r, pl.ds(base, 16)]
            key = _keyxform(lax.bitcast_convert_type(x, jnp.int32))
            offt = _append(cand_t0, offt, key, key >= lts)
            offb = _append(cand_b0, offb, key, key <= ubs)
            return offt, offb

        offt, offb = plsc.parallel_loop(
            0, NVREG, 1, unroll=8, carry=(_c(0), _c(0)))(s2_body)
        nt = offt[0]
        nb = offb[0]

        def refine(bufs, n0, p1byte, k0, is_top):
            p = p1byte ^ jnp.int32(128)
            k = k0
            n = n0
            for lvl in (2, 3, 4):
                src = bufs[lvl % 2]
                dst = bufs[1 - lvl % 2]
                shift_pref = 32 - 8 * (lvl - 1)
                shift_byte = 32 - 8 * lvl
                _clear_hist(hist)
                ps = _splat(p)
                ns = _splat(n)
                trip = (n + jnp.int32(15)) // jnp.int32(16)

                def hbody(i, _sp=shift_pref, _sb=shift_byte, _ps=ps, _ns=ns,
                          _src=src):
                    base = i * 16
                    ch = _src[pl.ds(base, 16)]
                    valid = (iota + _splat(base)) < _ns
                    pref = lax.shift_right_logical(ch, _c(_sp))
                    m_eq = (pref == _ps) & valid
                    byt = lax.shift_right_logical(ch, _c(_sb)) & _c(0xFF)
                    plsc.addupdate_scatter(hist, [byt], ones, mask=m_eq)

                plsc.parallel_loop(0, trip, 1, unroll=4)(hbody)
                if is_top:
                    b, g = _select_top(hist, k)
                else:
                    b, g = _select_bot(hist, k)
                k = k - g
                p = lax.shift_left(p, jnp.int32(8)) | b
                if is_top:
                    bound = lax.shift_left(p, jnp.int32(shift_byte))
                else:
                    bound = lax.shift_left(p + jnp.int32(1),
                                           jnp.int32(shift_byte)) - jnp.int32(1)
                bnd = _splat(bound)

                def fbody(i, off, _bnd=bnd, _ns=ns, _src=src, _dst=dst,
                          _top=is_top):
                    base = i * 16
                    ch = _src[pl.ds(base, 16)]
                    valid = (iota + _splat(base)) < _ns
                    m = ((ch >= _bnd) if _top else (ch <= _bnd)) & valid
                    return _append(_dst, off, ch, m)

                offv = plsc.parallel_loop(
                    0, trip, 1, unroll=4, carry=_c(0))(fbody)
                n = offv[0]
            return p, k, n

        t_top, ties_t, ntf = refine(tbufs, nt, b1t, kt, True)
        t_bot, ties_b, nbf = refine(bbufs, nb, b1b, kb, False)

        neg = jnp.full((L,), I32MIN, jnp.int32)
        for j in range(8):
            sgb[pl.ds(16 * j, 16)] = neg
            slb[pl.ds(16 * j, 16)] = neg

        tts = _splat(t_top)
        tbs = _splat(t_bot)
        ft = tbufs[1]
        fb = bbufs[1]

        def gbody(i, off, _ns=_splat(ntf)):
            base = i * 16
            ch = ft[pl.ds(base, 16)]
            valid = (iota + _splat(base)) < _ns
            return _append(sgb, off, ch, (ch > tts) & valid)

        def lbody(i, off, _ns=_splat(nbf)):
            base = i * 16
            ch = fb[pl.ds(base, 16)]
            valid = (iota + _splat(base)) < _ns
            return _append(slb, off, ch, (ch < tbs) & valid)

        tript = (ntf + jnp.int32(15)) // jnp.int32(16)
        tripb = (nbf + jnp.int32(15)) // jnp.int32(16)
        plsc.parallel_loop(0, tript, 1, unroll=4, carry=_c(0))(gbody)
        plsc.parallel_loop(0, tripb, 1, unroll=4, carry=_c(0))(lbody)

        sg = _sort128_desc([sgb[pl.ds(16 * j, 16)] for j in range(8)])
        sl = _sort128_desc([slb[pl.ds(16 * j, 16)] for j in range(8)])
        for j in range(8):
            slb[pl.ds(16 * j, 16)] = sl[j]

        if True:
            ng = _splat(jnp.int32(K) - ties_t)
            tbv = _splat(ties_b)
            for base in (0, 16, 32, 48, 64, 80, 96, 112, 128, 144, 160, 176,
                         184):
                pvec = iota + _c(base)
                if base <= 96:
                    vt = sg[base // 16]
                    valt = jnp.where(pvec < ng, vt, tts)
                if base >= 96:
                    q = pvec - _c(K)
                    qm = jnp.maximum(q - tbv, _c(0))
                    vb = plsc.load_gather(slb, [qm])
                    valb = jnp.where(q < tbv, tbs, vb)
                if base < 96:
                    val = valt
                elif base == 96:
                    val = jnp.where(pvec < _c(K), valt, valb)
                else:
                    val = valb
                stage[r, pl.ds(base, 16)] = lax.bitcast_convert_type(
                    _keyxform(val), jnp.float32)

            pltpu.async_copy(stage.at[r], out_hbm.at[row], osem)
        return carry

    lax.fori_loop(0, 2, row_work, jnp.int32(0))
    for i in range(2):
        pltpu.make_async_copy(stage.at[i], out_hbm.at[wid * 2 + i], osem).wait()


_mesh = plsc.VectorSubcoreMesh(core_axis_name="c", subcore_axis_name="s")

_chowder = functools.partial(
    pl.kernel,
    out_type=jax.ShapeDtypeStruct((ROWS, 2 * K), jnp.float32),
    mesh=_mesh,
    compiler_params=pltpu.CompilerParams(needs_layout_passes=False),
    scratch_types=[
        pltpu.VMEM((2, N), jnp.float32),
        pltpu.VMEM((N + 16,), jnp.int32),
        pltpu.VMEM((N + 16,), jnp.int32),
        pltpu.VMEM((N + 16,), jnp.int32),
        pltpu.VMEM((N + 16,), jnp.int32),
        pltpu.VMEM((256,), jnp.int32),
        pltpu.VMEM((128,), jnp.int32),
        pltpu.VMEM((128,), jnp.int32),
        pltpu.VMEM((2, 2 * K), jnp.float32),
        pltpu.SemaphoreType.DMA,
        pltpu.SemaphoreType.DMA,
    ],
)(_chowder_body)


def kernel(features, chowder_scores):
    del features
    return _chowder(chowder_scores)


# speedup vs baseline: 1.1278x; 1.1177x over previous
import functools

import jax
import jax.numpy as jnp
from jax import lax
from jax.experimental import pallas as pl
from jax.experimental.pallas import tpu as pltpu
from jax.experimental.pallas import tpu_sc as plsc

ROWS = 64
N = 8192
K = 100
L = 16
NVREG = N // L
I32MIN = -2147483648


def _c(v):
    return jnp.full((L,), v, jnp.int32)


def _splat(s):
    return jnp.full((L,), s, jnp.int32)


def _sort16d(v):
    return lax.rev(lax.sort(v, dimension=0), (0,))


def _bmerge_desc(vs):
    if len(vs) == 1:
        return [_sort16d(vs[0])]
    half = len(vs) // 2
    his, los = [], []
    for i in range(half):
        his.append(jnp.maximum(vs[i], vs[i + half]))
        los.append(jnp.minimum(vs[i], vs[i + half]))
    return _bmerge_desc(his) + _bmerge_desc(los)


def _merge_desc(a, b):
    m = len(a)
    revb = [lax.rev(v, (0,)) for v in b[::-1]]
    his, los = [], []
    for i in range(m):
        his.append(jnp.maximum(a[i], revb[i]))
        los.append(jnp.minimum(a[i], revb[i]))
    return _bmerge_desc(his) + _bmerge_desc(los)


def _sort128_desc(vs):
    runs = [[_sort16d(v)] for v in vs]
    while len(runs) > 1:
        runs = [_merge_desc(runs[i], runs[i + 1]) for i in range(0, len(runs), 2)]
    return runs[0]


def _keyxform(b):
    s = lax.shift_right_arithmetic(b, _c(31))
    return b ^ (s & _c(0x7FFFFFFF))


def _append(ref, off_splat, x, m):
    mi = m.astype(jnp.int32)
    cs = plsc.cumsum(mi)
    idx = off_splat + cs - _c(1)
    plsc.store_scatter(ref, [idx], x, mask=m)
    return off_splat + plsc.all_reduce_population_count(m)


def _select_top(hist, k):
    tail = jnp.int32(0)
    cnt = _c(0)
    ksplat = _splat(k)
    for j in range(15, -1, -1):
        h = hist[pl.ds(16 * j, 16)]
        rc = lax.rev(plsc.cumsum(lax.rev(h, (0,))), (0,))
        suf = rc + _splat(tail)
        cnt = cnt + plsc.all_reduce_population_count(suf >= ksplat)
        tail = tail + jnp.sum(h)
    b = jnp.max(cnt) - jnp.int32(1)
    g = jnp.int32(0)
    bs = _splat(b)
    iota = lax.iota(jnp.int32, L)
    for j in range(16):
        h = hist[pl.ds(16 * j, 16)]
        ids = iota + _c(16 * j)
        g = g + jnp.sum(jnp.where(ids > bs, h, _c(0)))
    return b, g


def _select_bot(hist, k):
    head = jnp.int32(0)
    cnt = _c(0)
    ksplat = _splat(k)
    for j in range(16):
        h = hist[pl.ds(16 * j, 16)]
        pc = plsc.cumsum(h)
        pre = pc + _splat(head)
        cnt = cnt + plsc.all_reduce_population_count(pre >= ksplat)
        head = head + jnp.sum(h)
    b = jnp.int32(256) - jnp.max(cnt)
    lcount = jnp.int32(0)
    bs = _splat(b)
    iota = lax.iota(jnp.int32, L)
    for j in range(16):
        h = hist[pl.ds(16 * j, 16)]
        ids = iota + _c(16 * j)
        lcount = lcount + jnp.sum(jnp.where(ids < bs, h, _c(0)))
    return b, lcount


def _clear_hist(hist):
    z = _c(0)
    for j in range(16):
        hist[pl.ds(16 * j, 16)] = z


def _chowder_body(scores_hbm, out_hbm, rawb, cand_t0, cand_t1, cand_b0,
                  cand_b1, hist, sgb, slb, stage, dsem, osem):
    nc = 2
    wid = lax.axis_index("s") * nc + lax.axis_index("c")
    iota = lax.iota(jnp.int32, L)
    tbufs = [cand_t0, cand_t1]
    bbufs = [cand_b0, cand_b1]

    pltpu.async_copy(scores_hbm.at[wid * 2], rawb.at[0], dsem)

    def row_work(r, carry):
        row = wid * 2 + r
        pltpu.make_async_copy(scores_hbm.at[row], rawb.at[r], dsem).wait()

        @pl.when(r == 0)
        def _():
            pltpu.async_copy(scores_hbm.at[wid * 2 + 1], rawb.at[1], dsem)

        _clear_hist(hist)
        ones = _c(1)

        def p1_body(i):
            base = i * 16
            x = rawb[r, pl.ds(base, 16)]
            key = _keyxform(lax.bitcast_convert_type(x, jnp.int32))
            byte0 = lax.shift_right_logical(key, _c(24)) ^ _c(0x80)
            plsc.addupdate_scatter(hist, [byte0], ones)

        plsc.parallel_loop(0, NVREG, 1, unroll=8)(p1_body)

        b1t, g1 = _select_top(hist, jnp.int32(K))
        b1b, l1 = _select_bot(hist, jnp.int32(K))
        kt = jnp.int32(K) - g1
        kb = jnp.int32(K) - l1

        lt = lax.shift_left(b1t ^ jnp.int32(128), jnp.int32(24))
        ub = lax.shift_left((b1b ^ jnp.int32(128)) + jnp.int32(1),
                            jnp.int32(24)) - jnp.int32(1)
        lts, ubs = _splat(lt), _splat(ub)

        negv = jnp.full((L,), I32MIN, jnp.int32)
        posv = jnp.full((L,), 2147483647, jnp.int32)
        for j in range(16):
            cand_t0[pl.ds(16 * j, 16)] = negv
            cand_b0[pl.ds(16 * j, 16)] = posv

        def s2_body(i, offs):
            offt, offb = offs
            base = i * 16
            x = rawb[r, pl.ds(base, 16)]
            key = _keyxform(lax.bitcast_convert_type(x, jnp.int32))
            offt = _append(cand_t0, offt, key, key >= lts)
            offb = _append(cand_b0, offb, key, key <= ubs)
            return offt, offb

        offt, offb = plsc.parallel_loop(
            0, NVREG, 1, unroll=8, carry=(_c(0), _c(0)))(s2_body)
        nt = offt[0]
        nb = offb[0]

        def slow_path():
            def refine(bufs, n0, p1byte, k0, is_top):
                p = p1byte ^ jnp.int32(128)
                k = k0
                n = n0
                for lvl in (2, 3, 4):
                    src = bufs[lvl % 2]
                    dst = bufs[1 - lvl % 2]
                    shift_pref = 32 - 8 * (lvl - 1)
                    shift_byte = 32 - 8 * lvl
                    _clear_hist(hist)
                    ps = _splat(p)
                    ns = _splat(n)
                    trip = (n + jnp.int32(15)) // jnp.int32(16)

                    def hbody(i, _sp=shift_pref, _sb=shift_byte, _ps=ps, _ns=ns,
                              _src=src):
                        base = i * 16
                        ch = _src[pl.ds(base, 16)]
                        valid = (iota + _splat(base)) < _ns
                        pref = lax.shift_right_logical(ch, _c(_sp))
                        m_eq = (pref == _ps) & valid
                        byt = lax.shift_right_logical(ch, _c(_sb)) & _c(0xFF)
                        plsc.addupdate_scatter(hist, [byt], ones, mask=m_eq)

                    plsc.parallel_loop(0, trip, 1, unroll=4)(hbody)
                    if is_top:
                        b, g = _select_top(hist, k)
                    else:
                        b, g = _select_bot(hist, k)
                    k = k - g
                    p = lax.shift_left(p, jnp.int32(8)) | b
                    if is_top:
                        bound = lax.shift_left(p, jnp.int32(shift_byte))
                    else:
                        bound = lax.shift_left(p + jnp.int32(1),
                                               jnp.int32(shift_byte)) - jnp.int32(1)
                    bnd = _splat(bound)

                    def fbody(i, off, _bnd=bnd, _ns=ns, _src=src, _dst=dst,
                              _top=is_top):
                        base = i * 16
                        ch = _src[pl.ds(base, 16)]
                        valid = (iota + _splat(base)) < _ns
                        m = ((ch >= _bnd) if _top else (ch <= _bnd)) & valid
                        return _append(_dst, off, ch, m)

                    offv = plsc.parallel_loop(
                        0, trip, 1, unroll=4, carry=_c(0))(fbody)
                    n = offv[0]
                return p, k, n

            t_top, ties_t, ntf = refine(tbufs, nt, b1t, kt, True)
            t_bot, ties_b, nbf = refine(bbufs, nb, b1b, kb, False)

            neg = jnp.full((L,), I32MIN, jnp.int32)
            for j in range(8):
                sgb[pl.ds(16 * j, 16)] = neg
                slb[pl.ds(16 * j, 16)] = neg

            tts = _splat(t_top)
            tbs = _splat(t_bot)
            ft = tbufs[1]
            fb = bbufs[1]

            def gbody(i, off, _ns=_splat(ntf)):
                base = i * 16
                ch = ft[pl.ds(base, 16)]
                valid = (iota + _splat(base)) < _ns
                return _append(sgb, off, ch, (ch > tts) & valid)

            def lbody(i, off, _ns=_splat(nbf)):
                base = i * 16
                ch = fb[pl.ds(base, 16)]
                valid = (iota + _splat(base)) < _ns
                return _append(slb, off, ch, (ch < tbs) & valid)

            tript = (ntf + jnp.int32(15)) // jnp.int32(16)
            tripb = (nbf + jnp.int32(15)) // jnp.int32(16)
            plsc.parallel_loop(0, tript, 1, unroll=4, carry=_c(0))(gbody)
            plsc.parallel_loop(0, tripb, 1, unroll=4, carry=_c(0))(lbody)

            sg = _sort128_desc([sgb[pl.ds(16 * j, 16)] for j in range(8)])
            sl = _sort128_desc([slb[pl.ds(16 * j, 16)] for j in range(8)])
            for j in range(8):
                slb[pl.ds(16 * j, 16)] = sl[j]

            if True:
                ng = _splat(jnp.int32(K) - ties_t)
                tbv = _splat(ties_b)
                for base in (0, 16, 32, 48, 64, 80, 96, 112, 128, 144, 160, 176,
                             184):
                    pvec = iota + _c(base)
                    if base <= 96:
                        vt = sg[base // 16]
                        valt = jnp.where(pvec < ng, vt, tts)
                    if base >= 96:
                        q = pvec - _c(K)
                        qm = jnp.maximum(q - tbv, _c(0))
                        vb = plsc.load_gather(slb, [qm])
                        valb = jnp.where(q < tbv, tbs, vb)
                    if base < 96:
                        val = valt
                    elif base == 96:
                        val = jnp.where(pvec < _c(K), valt, valb)
                    else:
                        val = valb
                    stage[r, pl.ds(base, 16)] = lax.bitcast_convert_type(
                        _keyxform(val), jnp.float32)


        def fast_path():
            st = _sort128_desc([cand_t0[pl.ds(16 * j, 16)] for j in range(16)])
            sb = _sort128_desc([cand_b0[pl.ds(16 * j, 16)] for j in range(16)])
            for j in range(16):
                slb[pl.ds(16 * j, 16)] = sb[j]

            def emit(v, base):
                stage[r, pl.ds(base, 16)] = lax.bitcast_convert_type(
                    _keyxform(v), jnp.float32)

            for base in (0, 16, 32, 48, 64, 80):
                emit(st[base // 16], base)
            pvec = iota + _c(96)
            vb96 = slb[pl.ds(152, 16)]
            emit(jnp.where(pvec < _c(K), st[6], vb96), 96)
            for base, src in ((112, 168), (128, 184), (144, 200), (160, 216),
                              (176, 232), (184, 240)):
                emit(slb[pl.ds(src, 16)], base)

        fast = jnp.logical_and(nt <= jnp.int32(256), nb <= jnp.int32(256))
        lax.cond(fast, fast_path, slow_path)
        pltpu.async_copy(stage.at[r], out_hbm.at[row], osem)
        return carry

    lax.fori_loop(0, 2, row_work, jnp.int32(0))
    for i in range(2):
        pltpu.make_async_copy(stage.at[i], out_hbm.at[wid * 2 + i], osem).wait()


_mesh = plsc.VectorSubcoreMesh(core_axis_name="c", subcore_axis_name="s")

_chowder = functools.partial(
    pl.kernel,
    out_type=jax.ShapeDtypeStruct((ROWS, 2 * K), jnp.float32),
    mesh=_mesh,
    compiler_params=pltpu.CompilerParams(needs_layout_passes=False),
    scratch_types=[
        pltpu.VMEM((2, N), jnp.float32),
        pltpu.VMEM((N + 16,), jnp.int32),
        pltpu.VMEM((N + 16,), jnp.int32),
        pltpu.VMEM((N + 16,), jnp.int32),
        pltpu.VMEM((N + 16,), jnp.int32),
        pltpu.VMEM((256,), jnp.int32),
        pltpu.VMEM((128,), jnp.int32),
        pltpu.VMEM((256,), jnp.int32),
        pltpu.VMEM((2, 2 * K), jnp.float32),
        pltpu.SemaphoreType.DMA,
        pltpu.SemaphoreType.DMA,
    ],
)(_chowder_body)


def kernel(features, chowder_scores):
    del features
    return _chowder(chowder_scores)
